# Initial kernel scaffold; baseline (speedup 1.0000x reference)
import jax
import jax.numpy as jnp
from jax.experimental import pallas as pl


def kernel(x, edge_attr, edge_index, W_res, W_gea, bn_gamma, bn_beta):
    raise NotImplementedError("write your pallas kernel here")


# speedup vs baseline: 6.5146x; 6.5146x over previous
import functools

import jax
import jax.numpy as jnp
from jax import lax
from jax.experimental import pallas as pl
from jax.experimental.pallas import tpu as pltpu
from jax.experimental.pallas import tpu_sc as plsc

EPS = 1e-6
BN_EPS = 1e-5

N = 10000
E = 320000
D = 128

NC = 2
NS = 16
NW = NC * NS
C = 128
CHUNKS = 79
EPW = CHUNKS * C
EPAD = NW * EPW
NPAD = 10240
RPT = NPAD // NS
ZB = RPT // C


def _sc_body(packed_hbm, ea_hbm, x_hbm, t_out, prop_out,
             t_sh, idx_v, ea_v, sig_v, rows_v, prop_l, sem):
    c = lax.axis_index("c")
    s = lax.axis_index("s")
    wid = c * NS + s

    zv = jnp.zeros((16,), jnp.float32)

    def zrow(r, carry):
        for k in range(8):
            rows_v[r, pl.ds(k * 16, 16)] = zv
        return carry
    lax.fori_loop(0, C, zrow, 0)

    def zprop(i, carry):
        prop_l[pl.ds(i * 16, 16)] = zv
        return carry
    lax.fori_loop(0, NPAD // 16, zprop, 0)

    for b in range(ZB):
        pltpu.sync_copy(rows_v, t_sh.at[pl.ds(s * RPT + b * C, C)])
    plsc.subcore_barrier()

    ebase = wid * CHUNKS

    def chunk(g, carry):
        pltpu.sync_copy(packed_hbm.at[ebase + g], idx_v)
        pltpu.sync_copy(ea_hbm.at[ebase + g], ea_v)
        src_idx = idx_v.at[0]
        dst_idx = idx_v.at[1]
        pltpu.async_copy(x_hbm.at[src_idx], rows_v, sem).wait()
        for k in range(8):
            ea = ea_v[pl.ds(k * 16, 16)]
            sg = 1.0 / (1.0 + jnp.exp(-ea))
            sig_v[pl.ds(k * 16, 16)] = sg
            di = idx_v[1, pl.ds(k * 16, 16)]
            plsc.addupdate_scatter(prop_l, [di], sg)
        def row(e, carry):
            bv = plsc.load_gather(sig_v, [jnp.full((16,), e, jnp.int32)])
            for k in range(8):
                sl = pl.ds(k * 16, 16)
                rows_v[e, sl] = rows_v[e, sl] * bv
            return carry
        lax.fori_loop(0, C, row, 0)
        pltpu.sync_copy(rows_v, t_sh.at[dst_idx], add=True)
        return carry

    lax.fori_loop(0, CHUNKS, chunk, 0)

    plsc.subcore_barrier()

    pltpu.sync_copy(prop_l, prop_out.at[pl.ds(wid * NPAD, NPAD)])
    pltpu.sync_copy(t_sh.at[pl.ds(s * RPT, RPT)],
                    t_out.at[pl.ds(c * NPAD + s * RPT, RPT)])


def _sc_scatter(packed, ea_ch, x):
    mesh = plsc.VectorSubcoreMesh(core_axis_name="c", subcore_axis_name="s")
    kern = pl.kernel(
        _sc_body,
        out_type=(
            jax.ShapeDtypeStruct((NC * NPAD, D), jnp.float32),
            jax.ShapeDtypeStruct((NW * NPAD,), jnp.float32),
        ),
        mesh=mesh,
        compiler_params=pltpu.CompilerParams(needs_layout_passes=False),
        scratch_types=(
            pltpu.VMEM_SHARED((NPAD, D), jnp.float32),
            pltpu.VMEM((2, C), jnp.int32),
            pltpu.VMEM((C,), jnp.float32),
            pltpu.VMEM((C,), jnp.float32),
            pltpu.VMEM((C, D), jnp.float32),
            pltpu.VMEM((NPAD,), jnp.float32),
            pltpu.SemaphoreType.DMA,
        ),
    )
    return kern(packed, ea_ch, x)


BN_BLK = 1024
BN_NB = NPAD // BN_BLK


def _tc_body(x_ref, t_ref, p_ref, wr_ref, wg_ref, g_ref, b_ref, o_ref,
             h_buf, acc):
    ph = pl.program_id(0)
    j = pl.program_id(1)

    @pl.when(ph == 0)
    def _compute():
        t = t_ref[0] + t_ref[1]
        prop = jnp.sum(p_ref[...], axis=0)
        s_mat = t / (prop[:, None] + EPS)
        h = (jnp.dot(x_ref[...], wr_ref[...],
                     preferred_element_type=jnp.float32)
             + jnp.dot(s_mat, wg_ref[...],
                       preferred_element_type=jnp.float32))
        h_buf[pl.ds(j * BN_BLK, BN_BLK), :] = h

        @pl.when(j == 0)
        def _init():
            acc[...] = jnp.zeros_like(acc)

        acc[0, :] += jnp.sum(h, axis=0)
        acc[1, :] += jnp.sum(h * h, axis=0)

    @pl.when(ph == 1)
    def _normalize():
        inv_n = 1.0 / float(N)
        mean = acc[0, :] * inv_n
        var = acc[1, :] * inv_n - mean * mean
        scale = g_ref[0, :] * lax.rsqrt(var + BN_EPS)
        shift = b_ref[0, :] - mean * scale
        h = h_buf[pl.ds(j * BN_BLK, BN_BLK), :]
        o_ref[...] = x_ref[...] + jnp.maximum(h * scale + shift, 0.0)


def _tc_dense(xp, t2, prop32, wr_t, wg_t, gamma, beta):
    return pl.pallas_call(
        _tc_body,
        grid=(2, BN_NB),
        in_specs=[
            pl.BlockSpec((BN_BLK, D), lambda p, j: (j, 0)),
            pl.BlockSpec((2, BN_BLK, D), lambda p, j: (0, j, 0)),
            pl.BlockSpec((NW, BN_BLK), lambda p, j: (0, j)),
            pl.BlockSpec((D, D), lambda p, j: (0, 0)),
            pl.BlockSpec((D, D), lambda p, j: (0, 0)),
            pl.BlockSpec((1, D), lambda p, j: (0, 0)),
            pl.BlockSpec((1, D), lambda p, j: (0, 0)),
        ],
        out_specs=pl.BlockSpec((BN_BLK, D), lambda p, j: (j * p, 0)),
        out_shape=jax.ShapeDtypeStruct((NPAD, D), jnp.float32),
        scratch_shapes=[
            pltpu.VMEM((NPAD, D), jnp.float32),
            pltpu.VMEM((2, D), jnp.float32),
        ],
    )(xp, t2, prop32, wr_t, wg_t, gamma, beta)


@jax.jit
def kernel(x, edge_attr, edge_index, W_res, W_gea, bn_gamma, bn_beta):
    src = edge_index[0].astype(jnp.int32)
    dst = edge_index[1].astype(jnp.int32)
    ea = edge_attr[:, 0]

    pad = EPAD - E
    srcp = jnp.concatenate([src, jnp.zeros((pad,), jnp.int32)])
    dstp = jnp.concatenate([dst, jnp.zeros((pad,), jnp.int32)])
    eap = jnp.concatenate([ea, jnp.full((pad,), -1e9, jnp.float32)])
    ea_ch = eap.reshape(EPAD // C, C)
    packed = jnp.stack([srcp, dstp], axis=0)
    packed = packed.reshape(2, EPAD // C, C).transpose(1, 0, 2)

    t_flat, prop_flat = _sc_scatter(packed, ea_ch, x)
    t2 = t_flat.reshape(NC, NPAD, D)
    prop32 = prop_flat.reshape(NW, NPAD)

    xp = jnp.concatenate([x, jnp.zeros((NPAD - N, D), jnp.float32)], axis=0)
    out = _tc_dense(xp, t2, prop32, W_res.T, W_gea.T,
                    bn_gamma.reshape(1, D), bn_beta.reshape(1, D))
    return out[:N]


# speedup vs baseline: 7.3065x; 1.1216x over previous
import functools

import jax
import jax.numpy as jnp
from jax import lax
from jax.experimental import pallas as pl
from jax.experimental.pallas import tpu as pltpu
from jax.experimental.pallas import tpu_sc as plsc

EPS = 1e-6
BN_EPS = 1e-5

N = 10000
E = 320000
D = 128

NC = 2
NS = 16
NW = NC * NS
C = 128
CHUNKS = 80
EPW = CHUNKS * C
EPAD = NW * EPW
NPAD = 10240
RPT = NPAD // NS
ZB = RPT // C


STAGE = 16
NSTAGE = CHUNKS // STAGE


def _sc_body(packed_hbm, ea_hbm, x_hbm, t_out, prop_out,
             t_sh, prop_sh, idx_st, ea_st, sig_v, zp_v, rows0_v, rows1_v,
             sem0, sem1):
    c = lax.axis_index("c")
    s = lax.axis_index("s")
    wid = c * NS + s

    zv = jnp.zeros((16,), jnp.float32)

    def zrow(r, carry):
        for k in range(8):
            rows0_v[r, pl.ds(k * 16, 16)] = zv
        return carry
    lax.fori_loop(0, C, zrow, 0)

    def zprop(i, carry):
        zp_v[pl.ds(i * 16, 16)] = zv
        return carry
    lax.fori_loop(0, RPT // 16, zprop, 0)

    for b in range(ZB):
        pltpu.sync_copy(rows0_v, t_sh.at[pl.ds(s * RPT + b * C, C)])
    pltpu.sync_copy(zp_v, prop_sh.at[pl.ds(s * RPT, RPT)])
    plsc.subcore_barrier()

    rows = (rows0_v, rows1_v)
    sems = (sem0, sem1)

    def gather(lcid, b):
        pltpu.async_copy(x_hbm.at[idx_st.at[2 * lcid]], rows[b], sems[b])

    def gather_wait(lcid, b):
        pltpu.make_async_copy(x_hbm.at[idx_st.at[2 * lcid]], rows[b],
                              sems[b]).wait()

    def do_chunk(lcid, rows_b):
        for k in range(8):
            ea = ea_st[lcid, pl.ds(k * 16, 16)]
            sg = 1.0 / (1.0 + jnp.exp(-ea))
            sig_v[pl.ds(k * 16, 16)] = sg
        pltpu.sync_copy(sig_v, prop_sh.at[idx_st.at[2 * lcid + 1]], add=True)
        def row(e, carry):
            bv = plsc.load_gather(sig_v, [jnp.full((16,), e, jnp.int32)])
            for k in range(8):
                sl = pl.ds(k * 16, 16)
                rows_b[e, sl] = rows_b[e, sl] * bv
            return carry
        lax.fori_loop(0, C, row, 0, unroll=4)
        pltpu.sync_copy(rows_b, t_sh.at[idx_st.at[2 * lcid + 1]], add=True)

    for st in range(NSTAGE):
        pltpu.sync_copy(packed_hbm.at[wid, pl.ds(st * 2 * STAGE, 2 * STAGE)],
                        idx_st)
        pltpu.sync_copy(ea_hbm.at[wid, pl.ds(st * STAGE, STAGE)], ea_st)

        gather(0, 0)

        def pair(h, carry):
            for b in range(2):
                lcid = h * 2 + b
                nxt = lcid + 1

                @pl.when(nxt < STAGE)
                def _prefetch():
                    gather(nxt, 1 - b)

                gather_wait(lcid, b)
                do_chunk(lcid, rows[b])
            return carry

        lax.fori_loop(0, STAGE // 2, pair, 0)

    plsc.subcore_barrier()

    pltpu.sync_copy(prop_sh.at[pl.ds(s * RPT, RPT)],
                    prop_out.at[pl.ds(c * NPAD + s * RPT, RPT)])
    pltpu.sync_copy(t_sh.at[pl.ds(s * RPT, RPT)],
                    t_out.at[pl.ds(c * NPAD + s * RPT, RPT)])


def _sc_scatter(packed, ea_ch, x):
    mesh = plsc.VectorSubcoreMesh(core_axis_name="c", subcore_axis_name="s")
    kern = pl.kernel(
        _sc_body,
        out_type=(
            jax.ShapeDtypeStruct((NC * NPAD, D), jnp.float32),
            jax.ShapeDtypeStruct((NC * NPAD,), jnp.float32),
        ),
        mesh=mesh,
        compiler_params=pltpu.CompilerParams(needs_layout_passes=False),
        scratch_types=(
            pltpu.VMEM_SHARED((NPAD, D), jnp.float32),
            pltpu.VMEM_SHARED((NPAD,), jnp.float32),
            pltpu.VMEM((2 * STAGE, C), jnp.int32),
            pltpu.VMEM((STAGE, C), jnp.float32),
            pltpu.VMEM((C,), jnp.float32),
            pltpu.VMEM((RPT,), jnp.float32),
            pltpu.VMEM((C, D), jnp.float32),
            pltpu.VMEM((C, D), jnp.float32),
            pltpu.SemaphoreType.DMA,
            pltpu.SemaphoreType.DMA,
        ),
    )
    return kern(packed, ea_ch, x)


BN_BLK = 1024
BN_NB = NPAD // BN_BLK


def _tc_body(x_ref, t_ref, p_ref, wr_ref, wg_ref, g_ref, b_ref, o_ref,
             h_buf, acc):
    ph = pl.program_id(0)
    j = pl.program_id(1)

    @pl.when(ph == 0)
    def _compute():
        t = t_ref[0] + t_ref[1]
        prop = jnp.sum(p_ref[...], axis=0)
        s_mat = t / (prop[:, None] + EPS)
        h = (jnp.dot(x_ref[...], wr_ref[...],
                     preferred_element_type=jnp.float32)
             + jnp.dot(s_mat, wg_ref[...],
                       preferred_element_type=jnp.float32))
        h_buf[pl.ds(j * BN_BLK, BN_BLK), :] = h

        @pl.when(j == 0)
        def _init():
            acc[...] = jnp.zeros_like(acc)

        acc[0, :] += jnp.sum(h, axis=0)
        acc[1, :] += jnp.sum(h * h, axis=0)

    @pl.when(ph == 1)
    def _normalize():
        inv_n = 1.0 / float(N)
        mean = acc[0, :] * inv_n
        var = acc[1, :] * inv_n - mean * mean
        scale = g_ref[0, :] * lax.rsqrt(var + BN_EPS)
        shift = b_ref[0, :] - mean * scale
        h = h_buf[pl.ds(j * BN_BLK, BN_BLK), :]
        o_ref[...] = x_ref[...] + jnp.maximum(h * scale + shift, 0.0)


def _tc_dense(xp, t2, prop32, wr_t, wg_t, gamma, beta):
    return pl.pallas_call(
        _tc_body,
        grid=(2, BN_NB),
        in_specs=[
            pl.BlockSpec((BN_BLK, D), lambda p, j: (j, 0)),
            pl.BlockSpec((2, BN_BLK, D), lambda p, j: (0, j, 0)),
            pl.BlockSpec((NC, BN_BLK), lambda p, j: (0, j)),
            pl.BlockSpec((D, D), lambda p, j: (0, 0)),
            pl.BlockSpec((D, D), lambda p, j: (0, 0)),
            pl.BlockSpec((1, D), lambda p, j: (0, 0)),
            pl.BlockSpec((1, D), lambda p, j: (0, 0)),
        ],
        out_specs=pl.BlockSpec((BN_BLK, D), lambda p, j: (j * p, 0)),
        out_shape=jax.ShapeDtypeStruct((NPAD, D), jnp.float32),
        scratch_shapes=[
            pltpu.VMEM((NPAD, D), jnp.float32),
            pltpu.VMEM((2, D), jnp.float32),
        ],
    )(xp, t2, prop32, wr_t, wg_t, gamma, beta)


@jax.jit
def kernel(x, edge_attr, edge_index, W_res, W_gea, bn_gamma, bn_beta):
    src = edge_index[0].astype(jnp.int32)
    dst = edge_index[1].astype(jnp.int32)
    ea = edge_attr[:, 0]

    pad = EPAD - E
    srcp = jnp.concatenate([src, jnp.zeros((pad,), jnp.int32)])
    dstp = jnp.concatenate([dst, jnp.zeros((pad,), jnp.int32)])
    eap = jnp.concatenate([ea, jnp.full((pad,), -1e9, jnp.float32)])
    ea_ch = eap.reshape(NW, CHUNKS, C)
    packed = jnp.stack([srcp, dstp], axis=0)
    packed = packed.reshape(2, NW, CHUNKS, C).transpose(1, 2, 0, 3)
    packed = packed.reshape(NW, CHUNKS * 2, C)

    t_flat, prop_flat = _sc_scatter(packed, ea_ch, x)
    t2 = t_flat.reshape(NC, NPAD, D)
    prop32 = prop_flat.reshape(NC, NPAD)

    xp = jnp.concatenate([x, jnp.zeros((NPAD - N, D), jnp.float32)], axis=0)
    out = _tc_dense(xp, t2, prop32, W_res.T, W_gea.T,
                    bn_gamma.reshape(1, D), bn_beta.reshape(1, D))
    return out[:N]


# speedup vs baseline: 8.7625x; 1.1993x over previous
import jax
import jax.numpy as jnp
from jax import lax
from jax.experimental import pallas as pl
from jax.experimental.pallas import tpu as pltpu
from jax.experimental.pallas import tpu_sc as plsc

EPS = 1e-6
BN_EPS = 1e-5

N = 10000
E = 320000
D = 128
HD = 64
NPASS = 2

NC = 2
NS = 16
NW = NC * NS
C = 128
CHUNKS = 80
EPW = CHUNKS * C
EPAD = NW * EPW
NPAD = 10240
RPT = NPAD // NS
ZB = RPT // C

STAGE = 16
NSTAGE = CHUNKS // STAGE


def _sc_body(packed_hbm, ea_hbm, xh_hbm, t_out, prop_out,
             x_sh, t_sh, prop_sh, idx_st, ea_st, sig_v, zp_v,
             rows0_v, rows1_v, sem0, sem1):
    c = lax.axis_index("c")
    s = lax.axis_index("s")
    wid = c * NS + s

    zv = jnp.zeros((16,), jnp.float32)

    def zprop(i, carry):
        zp_v[pl.ds(i * 16, 16)] = zv
        return carry
    lax.fori_loop(0, RPT // 16, zprop, 0)
    pltpu.sync_copy(zp_v, prop_sh.at[pl.ds(s * RPT, RPT)])

    def zrow(r, carry):
        for k in range(HD // 16):
            rows0_v[r, pl.ds(k * 16, 16)] = zv
        return carry

    rows = (rows0_v, rows1_v)
    sems = (sem0, sem1)

    for p in range(NPASS):
        pltpu.sync_copy(xh_hbm.at[pl.ds(p * NPAD + s * RPT, RPT)],
                        x_sh.at[pl.ds(s * RPT, RPT)])
        lax.fori_loop(0, C, zrow, 0)
        for b in range(ZB):
            pltpu.sync_copy(rows0_v, t_sh.at[pl.ds(s * RPT + b * C, C)])
        plsc.subcore_barrier()

        for st in range(NSTAGE):
            pltpu.sync_copy(
                packed_hbm.at[wid, pl.ds(st * 2 * STAGE, 2 * STAGE)],
                idx_st)
            pltpu.sync_copy(ea_hbm.at[wid, pl.ds(st * STAGE, STAGE)], ea_st)

            pltpu.async_copy(x_sh.at[idx_st.at[0]], rows[0], sems[0])

            def pair(h, carry):
                for b in range(2):
                    lcid = h * 2 + b
                    nxt = lcid + 1

                    @pl.when(nxt < STAGE)
                    def _prefetch():
                        pltpu.async_copy(x_sh.at[idx_st.at[2 * nxt]],
                                         rows[1 - b], sems[1 - b])

                    pltpu.make_async_copy(x_sh.at[idx_st.at[2 * lcid]],
                                          rows[b], sems[b]).wait()

                    for k in range(8):
                        ea = ea_st[lcid, pl.ds(k * 16, 16)]
                        sg = 1.0 / (1.0 + jnp.exp(-ea))
                        sig_v[pl.ds(k * 16, 16)] = sg
                    if p == 0:
                        pltpu.sync_copy(sig_v,
                                        prop_sh.at[idx_st.at[2 * lcid + 1]],
                                        add=True)

                    def row(e, carry2):
                        bv = plsc.load_gather(
                            sig_v, [jnp.full((16,), e, jnp.int32)])
                        for k in range(HD // 16):
                            sl = pl.ds(k * 16, 16)
                            rows[b][e, sl] = rows[b][e, sl] * bv
                        return carry2
                    lax.fori_loop(0, C, row, 0, unroll=4)

                    pltpu.sync_copy(rows[b],
                                    t_sh.at[idx_st.at[2 * lcid + 1]],
                                    add=True)
                return carry

            lax.fori_loop(0, STAGE // 2, pair, 0)

        plsc.subcore_barrier()

        pltpu.sync_copy(t_sh.at[pl.ds(s * RPT, RPT)],
                        t_out.at[pl.ds((p * NC + c) * NPAD + s * RPT, RPT)])
        if p == 0:
            pltpu.sync_copy(prop_sh.at[pl.ds(s * RPT, RPT)],
                            prop_out.at[pl.ds(c * NPAD + s * RPT, RPT)])
        plsc.subcore_barrier()


def _sc_scatter(packed, ea_ch, xh):
    mesh = plsc.VectorSubcoreMesh(core_axis_name="c", subcore_axis_name="s")
    kern = pl.kernel(
        _sc_body,
        out_type=(
            jax.ShapeDtypeStruct((NPASS * NC * NPAD, HD), jnp.float32),
            jax.ShapeDtypeStruct((NC * NPAD,), jnp.float32),
        ),
        mesh=mesh,
        compiler_params=pltpu.CompilerParams(needs_layout_passes=False),
        scratch_types=(
            pltpu.VMEM_SHARED((NPAD, HD), jnp.float32),
            pltpu.VMEM_SHARED((NPAD, HD), jnp.float32),
            pltpu.VMEM_SHARED((NPAD,), jnp.float32),
            pltpu.VMEM((2 * STAGE, C), jnp.int32),
            pltpu.VMEM((STAGE, C), jnp.float32),
            pltpu.VMEM((C,), jnp.float32),
            pltpu.VMEM((RPT,), jnp.float32),
            pltpu.VMEM((C, HD), jnp.float32),
            pltpu.VMEM((C, HD), jnp.float32),
            pltpu.SemaphoreType.DMA,
            pltpu.SemaphoreType.DMA,
        ),
    )
    return kern(packed, ea_ch, xh)


BN_BLK = 1024
BN_NB = NPAD // BN_BLK


def _tc_body(x_ref, t_ref, p_ref, wr_ref, wg_ref, g_ref, b_ref, o_ref,
             h_buf, acc):
    ph = pl.program_id(0)
    j = pl.program_id(1)

    @pl.when(ph == 0)
    def _compute():
        prop = jnp.sum(p_ref[...], axis=0)
        inv = 1.0 / (prop + EPS)
        t0 = (t_ref[0, 0] + t_ref[0, 1]) * inv[:, None]
        t1 = (t_ref[1, 0] + t_ref[1, 1]) * inv[:, None]
        wg = wg_ref[...]
        h = (jnp.dot(x_ref[...], wr_ref[...],
                     preferred_element_type=jnp.float32)
             + jnp.dot(t0, wg[:HD, :], preferred_element_type=jnp.float32)
             + jnp.dot(t1, wg[HD:, :], preferred_element_type=jnp.float32))
        h_buf[pl.ds(j * BN_BLK, BN_BLK), :] = h

        @pl.when(j == 0)
        def _init():
            acc[...] = jnp.zeros_like(acc)

        acc[0, :] += jnp.sum(h, axis=0)
        acc[1, :] += jnp.sum(h * h, axis=0)

    @pl.when(ph == 1)
    def _normalize():
        inv_n = 1.0 / float(N)
        mean = acc[0, :] * inv_n
        var = acc[1, :] * inv_n - mean * mean
        scale = g_ref[0, :] * lax.rsqrt(var + BN_EPS)
        shift = b_ref[0, :] - mean * scale
        h = h_buf[pl.ds(j * BN_BLK, BN_BLK), :]
        o_ref[...] = x_ref[...] + jnp.maximum(h * scale + shift, 0.0)


def _tc_dense(xp, t4, prop2, wr_t, wg_t, gamma, beta):
    return pl.pallas_call(
        _tc_body,
        grid=(2, BN_NB),
        in_specs=[
            pl.BlockSpec((BN_BLK, D), lambda p, j: (j, 0)),
            pl.BlockSpec((NPASS, NC, BN_BLK, HD), lambda p, j: (0, 0, j, 0)),
            pl.BlockSpec((NC, BN_BLK), lambda p, j: (0, j)),
            pl.BlockSpec((D, D), lambda p, j: (0, 0)),
            pl.BlockSpec((D, D), lambda p, j: (0, 0)),
            pl.BlockSpec((1, D), lambda p, j: (0, 0)),
            pl.BlockSpec((1, D), lambda p, j: (0, 0)),
        ],
        out_specs=pl.BlockSpec((BN_BLK, D), lambda p, j: (j * p, 0)),
        out_shape=jax.ShapeDtypeStruct((NPAD, D), jnp.float32),
        scratch_shapes=[
            pltpu.VMEM((NPAD, D), jnp.float32),
            pltpu.VMEM((2, D), jnp.float32),
        ],
    )(xp, t4, prop2, wr_t, wg_t, gamma, beta)


@jax.jit
def kernel(x, edge_attr, edge_index, W_res, W_gea, bn_gamma, bn_beta):
    src = edge_index[0].astype(jnp.int32)
    dst = edge_index[1].astype(jnp.int32)
    ea = edge_attr[:, 0]

    pad = EPAD - E
    srcp = jnp.concatenate([src, jnp.zeros((pad,), jnp.int32)])
    dstp = jnp.concatenate([dst, jnp.zeros((pad,), jnp.int32)])
    eap = jnp.concatenate([ea, jnp.full((pad,), -1e9, jnp.float32)])
    ea_ch = eap.reshape(NW, CHUNKS, C)
    packed = jnp.stack([srcp, dstp], axis=0)
    packed = packed.reshape(2, NW, CHUNKS, C).transpose(1, 2, 0, 3)
    packed = packed.reshape(NW, CHUNKS * 2, C)

    xp = jnp.concatenate([x, jnp.zeros((NPAD - N, D), jnp.float32)], axis=0)
    xh = xp.reshape(NPAD, NPASS, HD).transpose(1, 0, 2).reshape(-1, HD)

    t_flat, prop_flat = _sc_scatter(packed, ea_ch, xh)
    t4 = t_flat.reshape(NPASS, NC, NPAD, HD)
    prop2 = prop_flat.reshape(NC, NPAD)

    out = _tc_dense(xp, t4, prop2, W_res.T, W_gea.T,
                    bn_gamma.reshape(1, D), bn_beta.reshape(1, D))
    return out[:N]
